# baseline (speedup 1.0000x reference)
import functools

import jax
import jax.numpy as jnp
from jax import lax
from jax.experimental import pallas as pl
from jax.experimental.pallas import tpu as pltpu
from jax.experimental.pallas import tpu_sc as plsc

VOCAB = 1000000
D = 64
B = 16384 * 50
NW = 32
B_PER_W = B // NW
CHUNK = 128
N_CHUNKS = B_PER_W // CHUNK
SCALE = float(D) ** 0.5

_MESH = plsc.VectorSubcoreMesh(core_axis_name="c", subcore_axis_name="s")


@functools.partial(
    pl.kernel,
    out_type=jax.ShapeDtypeStruct((B, D), jnp.float32),
    mesh=_MESH,
    compiler_params=pltpu.CompilerParams(use_tc_tiling_on_sc=False),
    scratch_types=[
        pltpu.VMEM((N_CHUNKS, CHUNK), jnp.int32),
        pltpu.VMEM((CHUNK, D), jnp.float32),
        pltpu.VMEM((CHUNK, D), jnp.float32),
        pltpu.SemaphoreType.DMA,
        pltpu.SemaphoreType.DMA,
    ],
)
def _emb_lookup(x_hbm, table_hbm, out_hbm, idx_v, buf0, buf1, sem0, sem1):
    wid = lax.axis_index("s") * 2 + lax.axis_index("c")
    base = wid * B_PER_W

    pltpu.sync_copy(x_hbm.at[wid], idx_v)

    def scale_and_store(buf, chunk_id):
        def scale_row(r, carry):
            for c in range(D // 16):
                sl = pl.ds(c * 16, 16)
                buf[r, sl] = buf[r, sl] * SCALE
            return carry

        lax.fori_loop(0, CHUNK, scale_row, 0, unroll=2)
        pltpu.sync_copy(buf, out_hbm.at[pl.ds(base + chunk_id * CHUNK, CHUNK)])

    pltpu.async_copy(table_hbm.at[idx_v.at[0]], buf0, sem0)

    def pair(g, carry):
        c0 = 2 * g
        pltpu.async_copy(table_hbm.at[idx_v.at[c0 + 1]], buf1, sem1)
        pltpu.make_async_copy(table_hbm.at[idx_v.at[0]], buf0, sem0).wait()
        scale_and_store(buf0, c0)
        nxt = jnp.minimum(c0 + 2, N_CHUNKS - 1)
        pltpu.async_copy(table_hbm.at[idx_v.at[nxt]], buf0, sem0)
        pltpu.make_async_copy(table_hbm.at[idx_v.at[0]], buf1, sem1).wait()
        scale_and_store(buf1, c0 + 1)
        return carry

    lax.fori_loop(0, N_CHUNKS // 2, pair, 0)

    pltpu.make_async_copy(table_hbm.at[idx_v.at[0]], buf0, sem0).wait()


def kernel(x, table):
    xf = x.reshape(NW, N_CHUNKS, CHUNK).astype(jnp.int32)
    out = _emb_lookup(xf, table)
    return out.reshape(16384, 50, D)
